# Initial kernel scaffold; baseline (speedup 1.0000x reference)
import jax
import jax.numpy as jnp
from jax.experimental import pallas as pl


def kernel(f_s, f_t, idx, contrast_idx, memory_v1, memory_v2):
    raise NotImplementedError("write your pallas kernel here")


# speedup vs baseline: 3.6237x; 3.6237x over previous
import functools
import math

import jax
import jax.numpy as jnp
from jax import lax
from jax.experimental import pallas as pl
from jax.experimental.pallas import tpu as pltpu
from jax.experimental.pallas import tpu_sc as plsc

EPS = 1e-07
NCE_T = 0.07
NCE_M = 0.5
N_DATA = 100000
FEAT_DIM = 128
NCE_K = 1024
BATCH = 1024

NW = 32
L = 16
NE = BATCH * (NCE_K + 1)
EPW = NE // NW
SUB = 2048
NSUB = EPW // SUB
TAIL = EPW - NSUB * SUB
ROWS_T = 1024
GRID_M = (N_DATA + ROWS_T - 1) // ROWS_T

_SC_MESH = dict(core_axis_name="c", subcore_axis_name="s", num_cores=2,
                num_subcores=16)


def _wid():
    return lax.axis_index("s") * 2 + lax.axis_index("c")


def _mm_body(m1_ref, m2_ref, ft_ref, fs_ref, st_ref, ss_ref):
    dn = (((1,), (1,)), ((), ()))
    st_ref[...] = lax.dot_general(
        m1_ref[...], ft_ref[...], dn, precision=lax.Precision.HIGHEST,
        preferred_element_type=jnp.float32)
    ss_ref[...] = lax.dot_general(
        m2_ref[...], fs_ref[...], dn, precision=lax.Precision.HIGHEST,
        preferred_element_type=jnp.float32)


def _scores(mem1, mem2, f_t, f_s):
    return pl.pallas_call(
        _mm_body,
        grid=(GRID_M,),
        in_specs=[
            pl.BlockSpec((ROWS_T, FEAT_DIM), lambda i: (i, 0)),
            pl.BlockSpec((ROWS_T, FEAT_DIM), lambda i: (i, 0)),
            pl.BlockSpec((BATCH, FEAT_DIM), lambda i: (0, 0)),
            pl.BlockSpec((BATCH, FEAT_DIM), lambda i: (0, 0)),
        ],
        out_specs=[
            pl.BlockSpec((ROWS_T, BATCH), lambda i: (i, 0)),
            pl.BlockSpec((ROWS_T, BATCH), lambda i: (i, 0)),
        ],
        out_shape=[
            jax.ShapeDtypeStruct((N_DATA, BATCH), jnp.float32),
            jax.ShapeDtypeStruct((N_DATA, BATCH), jnp.float32),
        ],
    )(mem1, mem2, f_t, f_s)


CH = 128


def _ext_index_groups(cidx_v, ridx_v, lane_v, j0, ngroups):
    iota = lax.iota(jnp.int32, L)

    def body(g, _):
        j = j0 + g * L + iota
        c = cidx_v[pl.ds(g * L, L)]
        q = j >> 10
        b = q - (q * 1025 > j).astype(jnp.int32)
        b = b - (b * 1025 > j).astype(jnp.int32)
        ridx_v[pl.ds(g * L, L)] = c * 8 + (b >> 7)
        lane_v[pl.ds(g * L, L)] = b & 127
        return 0

    lax.fori_loop(0, ngroups, body, 0)


def _ext_extract(buf_v, lane_v, outv, lbase, obase, ngroups):
    iota = lax.iota(jnp.int32, L)
    inv_t = jnp.float32(1.0 / NCE_T)

    def body(g, _):
        rows = g * L + iota
        lanes = lane_v[pl.ds(lbase + g * L, L)]
        val = plsc.load_gather(buf_v, [rows, lanes])
        outv[pl.ds(obase + g * L, L)] = jnp.exp(val * inv_t)
        return 0

    lax.fori_loop(0, ngroups, body, 0)


def _ext_body(cidx_hbm, st128_hbm, ss128_hbm, outt_hbm, outs_hbm,
              cidx_v, ridx_v, lane_v, buf_v, outv, sem):
    base = _wid() * EPW

    def subtile(st, _):
        j0 = pl.multiple_of(base + st * SUB, 8)
        pltpu.sync_copy(cidx_hbm.at[pl.ds(j0, SUB)], cidx_v)
        _ext_index_groups(cidx_v, ridx_v, lane_v, j0, SUB // L)
        for src_hbm, out_hbm in ((st128_hbm, outt_hbm), (ss128_hbm, outs_hbm)):
            def chunk(cc, _):
                off = cc * CH
                pltpu.async_copy(
                    src_hbm.at[ridx_v.at[pl.ds(off, CH)]], buf_v, sem).wait()
                _ext_extract(buf_v, lane_v, outv, off, off, CH // L)
                return 0

            lax.fori_loop(0, SUB // CH, chunk, 0)
            pltpu.sync_copy(outv, out_hbm.at[pl.ds(j0, SUB)])
        return 0

    lax.fori_loop(0, NSUB, subtile, 0)

    j0 = pl.multiple_of(base + NSUB * SUB, 8)
    pltpu.sync_copy(cidx_hbm.at[pl.ds(j0, TAIL)], cidx_v.at[pl.ds(0, TAIL)])
    _ext_index_groups(cidx_v, ridx_v, lane_v, j0, TAIL // L)
    for src_hbm, out_hbm in ((st128_hbm, outt_hbm), (ss128_hbm, outs_hbm)):
        pltpu.async_copy(src_hbm.at[ridx_v.at[pl.ds(0, TAIL)]],
                         buf_v.at[pl.ds(0, TAIL)], sem).wait()
        _ext_extract(buf_v, lane_v, outv, 0, 0, TAIL // L)
        pltpu.sync_copy(outv.at[pl.ds(0, TAIL)], out_hbm.at[pl.ds(j0, TAIL)])


def _extract(flat_idx, st16, ss16):
    fn = pl.kernel(
        _ext_body,
        out_type=[
            jax.ShapeDtypeStruct((NE,), jnp.float32),
            jax.ShapeDtypeStruct((NE,), jnp.float32),
        ],
        mesh=plsc.VectorSubcoreMesh(**_SC_MESH),
        compiler_params=pltpu.CompilerParams(needs_layout_passes=False),
        scratch_types=[
            pltpu.VMEM((SUB,), jnp.int32),
            pltpu.VMEM((SUB,), jnp.int32),
            pltpu.VMEM((SUB,), jnp.int32),
            pltpu.VMEM((CH, FEAT_DIM), jnp.float32),
            pltpu.VMEM((SUB,), jnp.float32),
            pltpu.SemaphoreType.DMA,
        ],
    )
    return fn(flat_idx, st16, ss16)


_LB = 128
_NLB = BATCH // _LB


def _loss_body(outt_ref, outs_ref, loss_ref, acc_ref):
    p = pl.program_id(0)
    i = pl.program_id(1)

    @pl.when(jnp.logical_and(p == 0, i == 0))
    def _():
        acc_ref[0] = 0.0
        acc_ref[1] = 0.0
        acc_ref[2] = 0.0
        acc_ref[3] = 0.0

    @pl.when(p == 0)
    def _():
        acc_ref[0] = acc_ref[0] + jnp.sum(outt_ref[...])
        acc_ref[1] = acc_ref[1] + jnp.sum(outs_ref[...])

    @pl.when(p == 1)
    def _():
        n_all = jnp.float32(BATCH * (NCE_K + 1))
        z_t = acc_ref[0] / n_all * jnp.float32(N_DATA)
        z_s = acc_ref[1] / n_all * jnp.float32(N_DATA)
        c0 = jnp.float32(NCE_K * (1.0 / N_DATA))
        kiota = lax.broadcasted_iota(jnp.int32, (_LB, NCE_K + 1), 1)
        for ref, z, slot in ((outs_ref, z_s, 2), (outt_ref, z_t, 3)):
            x = ref[...] / z
            den = (x + c0) + jnp.float32(EPS)
            d1 = jnp.log(x / den)
            d0 = jnp.log(c0 / den)
            terms = jnp.where(kiota == 0, d1, d0)
            acc_ref[slot] = acc_ref[slot] + jnp.sum(terms)

        @pl.when(i == _NLB - 1)
        def _():
            loss_ref[...] = jnp.full(
                (1, 1), -(acc_ref[2] + acc_ref[3]) / jnp.float32(BATCH),
                jnp.float32)


def _loss(out_t2d, out_s2d):
    return pl.pallas_call(
        _loss_body,
        grid=(2, _NLB),
        in_specs=[
            pl.BlockSpec((_LB, NCE_K + 1), lambda p, i: (i, 0)),
            pl.BlockSpec((_LB, NCE_K + 1), lambda p, i: (i, 0)),
        ],
        out_specs=pl.BlockSpec((1, 1), lambda p, i: (0, 0)),
        out_shape=jax.ShapeDtypeStruct((1, 1), jnp.float32),
        scratch_shapes=[pltpu.SMEM((4,), jnp.float32)],
    )(out_t2d, out_s2d)


RPW = BATCH // NW


def _upd_one(rows_v, f_v, upd_v):
    def body(i, _):
        acc = jnp.zeros((L,), jnp.float32)
        for p_ in range(FEAT_DIM // L):
            sl = pl.ds(p_ * L, L)
            v = rows_v[i, sl] * jnp.float32(NCE_M) + \
                f_v[i, sl] * jnp.float32(1.0 - NCE_M)
            upd_v[i, sl] = v
            acc = acc + v * v
        s = jnp.sum(acc)
        sv = jnp.full((L,), s, jnp.float32)
        iv = plsc.bitcast(sv, jnp.int32)
        y = plsc.bitcast(jnp.int32(0x5F3759DF) - (iv >> 1), jnp.float32)
        for _n in range(4):
            y = y * (jnp.float32(1.5) - jnp.float32(0.5) * sv * y * y)
        for p_ in range(FEAT_DIM // L):
            sl = pl.ds(p_ * L, L)
            upd_v[i, sl] = upd_v[i, sl] * y
        return 0

    lax.fori_loop(0, RPW, body, 0)


def _upd_body(idx_hbm, win_hbm, fs_hbm, ft_hbm, m1_hbm, m2_hbm,
              new1_ref, new2_ref, idx_v, win_v, rows_v, f_v, upd_v, sem):
    base = _wid() * RPW
    pltpu.sync_copy(idx_hbm.at[pl.ds(base, RPW)], idx_v)
    pltpu.sync_copy(win_hbm.at[pl.ds(base, RPW)], win_v)
    for mem_hbm, f_hbm, new_ref in ((m1_hbm, fs_hbm, new1_ref),
                                    (m2_hbm, ft_hbm, new2_ref)):
        pltpu.async_copy(mem_hbm.at[idx_v], rows_v, sem).wait()
        pltpu.async_copy(f_hbm.at[win_v], f_v, sem).wait()
        _upd_one(rows_v, f_v, upd_v)
        pltpu.sync_copy(upd_v, new_ref.at[idx_v])


def _update(idx, winner, f_s, f_t, mem1, mem2, new1_ref, new2_ref):
    fn = pl.kernel(
        _upd_body,
        out_type=(),
        mesh=plsc.VectorSubcoreMesh(**_SC_MESH),
        compiler_params=pltpu.CompilerParams(needs_layout_passes=False),
        scratch_types=[
            pltpu.VMEM((RPW,), jnp.int32),
            pltpu.VMEM((RPW,), jnp.int32),
            pltpu.VMEM((RPW, FEAT_DIM), jnp.float32),
            pltpu.VMEM((RPW, FEAT_DIM), jnp.float32),
            pltpu.VMEM((RPW, FEAT_DIM), jnp.float32),
            pltpu.SemaphoreType.DMA,
        ],
    )
    fn(idx, winner, f_s, f_t, mem1, mem2, new1_ref, new2_ref)


def kernel(f_s, f_t, idx, contrast_idx, memory_v1, memory_v2):
    idx = idx.astype(jnp.int32)
    flat_idx = contrast_idx.reshape(-1).astype(jnp.int32)

    ar = jnp.arange(BATCH, dtype=jnp.int32)
    eq = idx[:, None] == idx[None, :]
    winner = jnp.max(jnp.where(eq, ar[None, :], -1), axis=1).astype(jnp.int32)

    st, ss = _scores(memory_v1, memory_v2, f_t, f_s)
    out_t_flat, out_s_flat = _extract(
        flat_idx, st.reshape(-1, FEAT_DIM), ss.reshape(-1, FEAT_DIM))
    loss = _loss(out_t_flat.reshape(BATCH, NCE_K + 1),
                 out_s_flat.reshape(BATCH, NCE_K + 1)).reshape((1,))

    new1_ref = jax.new_ref(memory_v1)
    new2_ref = jax.new_ref(memory_v2)
    _update(idx, winner, f_s, f_t, memory_v1, memory_v2, new1_ref, new2_ref)
    return loss, new1_ref[...], new2_ref[...]


# speedup vs baseline: 5.2909x; 1.4601x over previous
import functools
import math

import jax
import jax.numpy as jnp
from jax import lax
from jax.experimental import pallas as pl
from jax.experimental.pallas import tpu as pltpu
from jax.experimental.pallas import tpu_sc as plsc

EPS = 1e-07
NCE_T = 0.07
NCE_M = 0.5
N_DATA = 100000
FEAT_DIM = 128
NCE_K = 1024
BATCH = 1024

NW = 32
L = 16
NE = BATCH * (NCE_K + 1)
EPW = NE // NW
SUB = 2048
NSUB = EPW // SUB
TAIL = EPW - NSUB * SUB
ROWS_T = 1024
GRID_M = (N_DATA + ROWS_T - 1) // ROWS_T

_SC_MESH = dict(core_axis_name="c", subcore_axis_name="s", num_cores=2,
                num_subcores=16)


def _wid():
    return lax.axis_index("s") * 2 + lax.axis_index("c")


_BB = BATCH // FEAT_DIM


def _mm_body(m1_ref, m2_ref, ft_ref, fs_ref, st_ref, ss_ref):
    dn = (((1,), (1,)), ((), ()))
    st_ref[...] = lax.dot_general(
        m1_ref[...], ft_ref[...], dn, precision=lax.Precision.HIGHEST,
        preferred_element_type=jnp.float32).reshape(ROWS_T, _BB, FEAT_DIM)
    ss_ref[...] = lax.dot_general(
        m2_ref[...], fs_ref[...], dn, precision=lax.Precision.HIGHEST,
        preferred_element_type=jnp.float32).reshape(ROWS_T, _BB, FEAT_DIM)


def _scores(mem1, mem2, f_t, f_s):
    return pl.pallas_call(
        _mm_body,
        grid=(GRID_M,),
        in_specs=[
            pl.BlockSpec((ROWS_T, FEAT_DIM), lambda i: (i, 0)),
            pl.BlockSpec((ROWS_T, FEAT_DIM), lambda i: (i, 0)),
            pl.BlockSpec((BATCH, FEAT_DIM), lambda i: (0, 0)),
            pl.BlockSpec((BATCH, FEAT_DIM), lambda i: (0, 0)),
        ],
        out_specs=[
            pl.BlockSpec((ROWS_T, _BB, FEAT_DIM), lambda i: (i, 0, 0)),
            pl.BlockSpec((ROWS_T, _BB, FEAT_DIM), lambda i: (i, 0, 0)),
        ],
        out_shape=[
            jax.ShapeDtypeStruct((N_DATA, _BB, FEAT_DIM), jnp.float32),
            jax.ShapeDtypeStruct((N_DATA, _BB, FEAT_DIM), jnp.float32),
        ],
    )(mem1, mem2, f_t, f_s)


CH = 128


def _ext_index_groups(cidx_v, ridx_v, lane_v, j0, ngroups):
    iota = lax.iota(jnp.int32, L)

    def body(g, _):
        j = j0 + g * L + iota
        c = cidx_v[pl.ds(g * L, L)]
        q = j >> 10
        b = q - (q * 1025 > j).astype(jnp.int32)
        b = b - (b * 1025 > j).astype(jnp.int32)
        ridx_v[pl.ds(g * L, L)] = c * 8 + (b >> 7)
        lane_v[pl.ds(g * L, L)] = b & 127
        return 0

    lax.fori_loop(0, ngroups, body, 0)


def _ext_extract(buf_v, lane_v, outv, lbase, obase, ngroups):
    iota = lax.iota(jnp.int32, L)
    inv_t = jnp.float32(1.0 / NCE_T)

    def body(g, _):
        rows = g * L + iota
        lanes = lane_v[pl.ds(lbase + g * L, L)]
        val = plsc.load_gather(buf_v, [rows, lanes])
        outv[pl.ds(obase + g * L, L)] = jnp.exp(val * inv_t)
        return 0

    lax.fori_loop(0, ngroups, body, 0)


def _ext_body(cidx_hbm, st128_hbm, ss128_hbm, outt_hbm, outs_hbm,
              cidx_v, ridx_v, lane_v, buf_v, outv, sem):
    base = _wid() * EPW

    def subtile(st, _):
        j0 = pl.multiple_of(base + st * SUB, 8)
        pltpu.sync_copy(cidx_hbm.at[pl.ds(j0, SUB)], cidx_v)
        _ext_index_groups(cidx_v, ridx_v, lane_v, j0, SUB // L)
        for src_hbm, out_hbm in ((st128_hbm, outt_hbm), (ss128_hbm, outs_hbm)):
            def chunk(cc, _):
                off = cc * CH
                pltpu.async_copy(
                    src_hbm.at[ridx_v.at[pl.ds(off, CH)]], buf_v, sem).wait()
                _ext_extract(buf_v, lane_v, outv, off, off, CH // L)
                return 0

            lax.fori_loop(0, SUB // CH, chunk, 0)
            pltpu.sync_copy(outv, out_hbm.at[pl.ds(j0, SUB)])
        return 0

    lax.fori_loop(0, NSUB, subtile, 0)

    j0 = pl.multiple_of(base + NSUB * SUB, 8)
    pltpu.sync_copy(cidx_hbm.at[pl.ds(j0, TAIL)], cidx_v.at[pl.ds(0, TAIL)])
    _ext_index_groups(cidx_v, ridx_v, lane_v, j0, TAIL // L)
    for src_hbm, out_hbm in ((st128_hbm, outt_hbm), (ss128_hbm, outs_hbm)):
        pltpu.async_copy(src_hbm.at[ridx_v.at[pl.ds(0, TAIL)]],
                         buf_v.at[pl.ds(0, TAIL)], sem).wait()
        _ext_extract(buf_v, lane_v, outv, 0, 0, TAIL // L)
        pltpu.sync_copy(outv.at[pl.ds(0, TAIL)], out_hbm.at[pl.ds(j0, TAIL)])


def _extract(flat_idx, st16, ss16):
    fn = pl.kernel(
        _ext_body,
        out_type=[
            jax.ShapeDtypeStruct((NE,), jnp.float32),
            jax.ShapeDtypeStruct((NE,), jnp.float32),
        ],
        mesh=plsc.VectorSubcoreMesh(**_SC_MESH),
        compiler_params=pltpu.CompilerParams(needs_layout_passes=False),
        scratch_types=[
            pltpu.VMEM((SUB,), jnp.int32),
            pltpu.VMEM((SUB,), jnp.int32),
            pltpu.VMEM((SUB,), jnp.int32),
            pltpu.VMEM((CH, FEAT_DIM), jnp.float32),
            pltpu.VMEM((SUB,), jnp.float32),
            pltpu.SemaphoreType.DMA,
        ],
    )
    return fn(flat_idx, st16, ss16)


_LB = 128
_NLB = BATCH // _LB


def _loss_body(outt_ref, outs_ref, loss_ref, acc_ref):
    p = pl.program_id(0)
    i = pl.program_id(1)

    @pl.when(jnp.logical_and(p == 0, i == 0))
    def _():
        acc_ref[0] = 0.0
        acc_ref[1] = 0.0
        acc_ref[2] = 0.0
        acc_ref[3] = 0.0

    @pl.when(p == 0)
    def _():
        acc_ref[0] = acc_ref[0] + jnp.sum(outt_ref[...])
        acc_ref[1] = acc_ref[1] + jnp.sum(outs_ref[...])

    @pl.when(p == 1)
    def _():
        n_all = jnp.float32(BATCH * (NCE_K + 1))
        z_t = acc_ref[0] / n_all * jnp.float32(N_DATA)
        z_s = acc_ref[1] / n_all * jnp.float32(N_DATA)
        c0 = jnp.float32(NCE_K * (1.0 / N_DATA))
        kiota = lax.broadcasted_iota(jnp.int32, (_LB, NCE_K + 1), 1)
        for ref, z, slot in ((outs_ref, z_s, 2), (outt_ref, z_t, 3)):
            x = ref[...] / z
            den = (x + c0) + jnp.float32(EPS)
            d1 = jnp.log(x / den)
            d0 = jnp.log(c0 / den)
            terms = jnp.where(kiota == 0, d1, d0)
            acc_ref[slot] = acc_ref[slot] + jnp.sum(terms)

        @pl.when(i == _NLB - 1)
        def _():
            loss_ref[...] = jnp.full(
                (1, 1), -(acc_ref[2] + acc_ref[3]) / jnp.float32(BATCH),
                jnp.float32)


def _loss(out_t2d, out_s2d):
    return pl.pallas_call(
        _loss_body,
        grid=(2, _NLB),
        in_specs=[
            pl.BlockSpec((_LB, NCE_K + 1), lambda p, i: (i, 0)),
            pl.BlockSpec((_LB, NCE_K + 1), lambda p, i: (i, 0)),
        ],
        out_specs=pl.BlockSpec((1, 1), lambda p, i: (0, 0)),
        out_shape=jax.ShapeDtypeStruct((1, 1), jnp.float32),
        scratch_shapes=[pltpu.SMEM((4,), jnp.float32)],
    )(out_t2d, out_s2d)


RPW = BATCH // NW


def _upd_one(rows_v, f_v, upd_v):
    def body(i, _):
        acc = jnp.zeros((L,), jnp.float32)
        for p_ in range(FEAT_DIM // L):
            sl = pl.ds(p_ * L, L)
            v = rows_v[i, sl] * jnp.float32(NCE_M) + \
                f_v[i, sl] * jnp.float32(1.0 - NCE_M)
            upd_v[i, sl] = v
            acc = acc + v * v
        s = jnp.sum(acc)
        sv = jnp.full((L,), s, jnp.float32)
        iv = plsc.bitcast(sv, jnp.int32)
        y = plsc.bitcast(jnp.int32(0x5F3759DF) - (iv >> 1), jnp.float32)
        for _n in range(4):
            y = y * (jnp.float32(1.5) - jnp.float32(0.5) * sv * y * y)
        for p_ in range(FEAT_DIM // L):
            sl = pl.ds(p_ * L, L)
            upd_v[i, sl] = upd_v[i, sl] * y
        return 0

    lax.fori_loop(0, RPW, body, 0)


def _upd_body(idx_hbm, win_hbm, fs_hbm, ft_hbm, m1_hbm, m2_hbm,
              new1_ref, new2_ref, idx_v, win_v, rows_v, f_v, upd_v, sem):
    base = _wid() * RPW
    pltpu.sync_copy(idx_hbm.at[pl.ds(base, RPW)], idx_v)
    pltpu.sync_copy(win_hbm.at[pl.ds(base, RPW)], win_v)
    for mem_hbm, f_hbm, new_ref in ((m1_hbm, fs_hbm, new1_ref),
                                    (m2_hbm, ft_hbm, new2_ref)):
        pltpu.async_copy(mem_hbm.at[idx_v], rows_v, sem).wait()
        pltpu.async_copy(f_hbm.at[win_v], f_v, sem).wait()
        _upd_one(rows_v, f_v, upd_v)
        pltpu.sync_copy(upd_v, new_ref.at[idx_v])


def _update(idx, winner, f_s, f_t, mem1, mem2, new1_ref, new2_ref):
    fn = pl.kernel(
        _upd_body,
        out_type=(),
        mesh=plsc.VectorSubcoreMesh(**_SC_MESH),
        compiler_params=pltpu.CompilerParams(needs_layout_passes=False),
        scratch_types=[
            pltpu.VMEM((RPW,), jnp.int32),
            pltpu.VMEM((RPW,), jnp.int32),
            pltpu.VMEM((RPW, FEAT_DIM), jnp.float32),
            pltpu.VMEM((RPW, FEAT_DIM), jnp.float32),
            pltpu.VMEM((RPW, FEAT_DIM), jnp.float32),
            pltpu.SemaphoreType.DMA,
        ],
    )
    fn(idx, winner, f_s, f_t, mem1, mem2, new1_ref, new2_ref)


def kernel(f_s, f_t, idx, contrast_idx, memory_v1, memory_v2):
    idx = idx.astype(jnp.int32)
    flat_idx = contrast_idx.reshape(-1).astype(jnp.int32)

    ar = jnp.arange(BATCH, dtype=jnp.int32)
    eq = idx[:, None] == idx[None, :]
    winner = jnp.max(jnp.where(eq, ar[None, :], -1), axis=1).astype(jnp.int32)

    st, ss = _scores(memory_v1, memory_v2, f_t, f_s)
    st128 = st.reshape(N_DATA * _BB, FEAT_DIM)
    ss128 = ss.reshape(N_DATA * _BB, FEAT_DIM)
    out_t_flat, out_s_flat = _extract(flat_idx, st128, ss128)
    loss = _loss(out_t_flat.reshape(BATCH, NCE_K + 1),
                 out_s_flat.reshape(BATCH, NCE_K + 1)).reshape((1,))

    new1_ref = jax.new_ref(memory_v1)
    new2_ref = jax.new_ref(memory_v2)
    _update(idx, winner, f_s, f_t, memory_v1, memory_v2, new1_ref, new2_ref)
    return loss, new1_ref[...], new2_ref[...]


# speedup vs baseline: 8.0661x; 1.5245x over previous
import functools
import math

import jax
import jax.numpy as jnp
from jax import lax
from jax.experimental import pallas as pl
from jax.experimental.pallas import tpu as pltpu
from jax.experimental.pallas import tpu_sc as plsc

EPS = 1e-07
NCE_T = 0.07
NCE_M = 0.5
N_DATA = 100000
FEAT_DIM = 128
NCE_K = 1024
BATCH = 1024

NW = 32
L = 16
NE = BATCH * (NCE_K + 1)
EPW = NE // NW
SUB = 2048
NSUB = EPW // SUB
TAIL = EPW - NSUB * SUB
ROWS_T = 1024
GRID_M = (N_DATA + ROWS_T - 1) // ROWS_T

_SC_MESH = dict(core_axis_name="c", subcore_axis_name="s", num_cores=2,
                num_subcores=16)


def _wid():
    return lax.axis_index("s") * 2 + lax.axis_index("c")


_BB = BATCH // FEAT_DIM


def _mm_body(m_ref, f_ref, st_ref, cp_ref):
    dn = (((1,), (1,)), ((), ()))
    st_ref[...] = lax.dot_general(
        m_ref[...], f_ref[...], dn, precision=lax.Precision.HIGHEST,
        preferred_element_type=jnp.float32).reshape(ROWS_T, _BB, FEAT_DIM)
    cp_ref[...] = m_ref[...]


def _scores(mem, f):
    return pl.pallas_call(
        _mm_body,
        grid=(GRID_M,),
        in_specs=[
            pl.BlockSpec((ROWS_T, FEAT_DIM), lambda i: (i, 0)),
            pl.BlockSpec((BATCH, FEAT_DIM), lambda i: (0, 0)),
        ],
        out_specs=[
            pl.BlockSpec((ROWS_T, _BB, FEAT_DIM), lambda i: (i, 0, 0)),
            pl.BlockSpec((ROWS_T, FEAT_DIM), lambda i: (i, 0)),
        ],
        out_shape=[
            jax.ShapeDtypeStruct((N_DATA, _BB, FEAT_DIM), jnp.float32),
            jax.ShapeDtypeStruct((N_DATA, FEAT_DIM), jnp.float32),
        ],
    )(mem, f)


CH = 128


def _ext_index_groups(cidx_v, ridx_v, lane_v, j0, ngroups):
    iota = lax.iota(jnp.int32, L)

    def body(g, _):
        j = j0 + g * L + iota
        c = cidx_v[pl.ds(g * L, L)]
        q = j >> 10
        b = q - (q * 1025 > j).astype(jnp.int32)
        b = b - (b * 1025 > j).astype(jnp.int32)
        ridx_v[pl.ds(g * L, L)] = c * 8 + (b >> 7)
        lane_v[pl.ds(g * L, L)] = b & 127
        return 0

    lax.fori_loop(0, ngroups, body, 0)


def _ext_extract(buf_v, lane_v, outv, lbase, obase, ngroups):
    iota = lax.iota(jnp.int32, L)
    inv_t = jnp.float32(1.0 / NCE_T)

    def body(g, _):
        rows = g * L + iota
        lanes = lane_v[pl.ds(lbase + g * L, L)]
        val = plsc.load_gather(buf_v, [rows, lanes])
        outv[pl.ds(obase + g * L, L)] = jnp.exp(val * inv_t)
        return 0

    lax.fori_loop(0, ngroups, body, 0)


NCH = SUB // CH


def _ext_body(cidx_hbm, s128_hbm, out_hbm,
              cidx_v, ridx_v, lane_v, buf0, buf1, outv, sem0, sem1):
    base = _wid() * EPW
    bufs = (buf0, buf1)
    sems = (sem0, sem1)

    def fire(cc, k):
        return pltpu.async_copy(
            s128_hbm.at[ridx_v.at[pl.ds(cc * CH, CH)]], bufs[k], sems[k])

    def subtile(st, _):
        j0 = pl.multiple_of(base + st * SUB, 8)
        pltpu.sync_copy(cidx_hbm.at[pl.ds(j0, SUB)], cidx_v)
        _ext_index_groups(cidx_v, ridx_v, lane_v, j0, SUB // L)
        cps = [fire(0, 0), fire(1, 1)]
        for cc in range(NCH):
            k = cc & 1
            cps[k].wait()
            _ext_extract(bufs[k], lane_v, outv, cc * CH, cc * CH, CH // L)
            if cc + 2 < NCH:
                cps[k] = fire(cc + 2, k)
        pltpu.sync_copy(outv, out_hbm.at[pl.ds(j0, SUB)])
        return 0

    lax.fori_loop(0, NSUB, subtile, 0)

    j0 = pl.multiple_of(base + NSUB * SUB, 8)
    pltpu.sync_copy(cidx_hbm.at[pl.ds(j0, TAIL)], cidx_v.at[pl.ds(0, TAIL)])
    _ext_index_groups(cidx_v, ridx_v, lane_v, j0, TAIL // L)
    pltpu.async_copy(s128_hbm.at[ridx_v.at[pl.ds(0, TAIL)]],
                     buf0.at[pl.ds(0, TAIL)], sem0).wait()
    _ext_extract(buf0, lane_v, outv, 0, 0, TAIL // L)
    pltpu.sync_copy(outv.at[pl.ds(0, TAIL)], out_hbm.at[pl.ds(j0, TAIL)])


def _extract(flat_idx, s128):
    fn = pl.kernel(
        _ext_body,
        out_type=jax.ShapeDtypeStruct((NE,), jnp.float32),
        mesh=plsc.VectorSubcoreMesh(**_SC_MESH),
        compiler_params=pltpu.CompilerParams(needs_layout_passes=False),
        scratch_types=[
            pltpu.VMEM((SUB,), jnp.int32),
            pltpu.VMEM((SUB,), jnp.int32),
            pltpu.VMEM((SUB,), jnp.int32),
            pltpu.VMEM((CH, FEAT_DIM), jnp.float32),
            pltpu.VMEM((CH, FEAT_DIM), jnp.float32),
            pltpu.VMEM((SUB,), jnp.float32),
            pltpu.SemaphoreType.DMA,
            pltpu.SemaphoreType.DMA,
        ],
    )
    return fn(flat_idx, s128)


_LB = 128
_NLB = BATCH // _LB


def _loss_body(outt_ref, outs_ref, loss_ref, acc_ref):
    p = pl.program_id(0)
    i = pl.program_id(1)

    @pl.when(jnp.logical_and(p == 0, i == 0))
    def _():
        acc_ref[0] = 0.0
        acc_ref[1] = 0.0
        acc_ref[2] = 0.0
        acc_ref[3] = 0.0

    @pl.when(p == 0)
    def _():
        acc_ref[0] = acc_ref[0] + jnp.sum(outt_ref[...])
        acc_ref[1] = acc_ref[1] + jnp.sum(outs_ref[...])

    @pl.when(p == 1)
    def _():
        n_all = jnp.float32(BATCH * (NCE_K + 1))
        z_t = acc_ref[0] / n_all * jnp.float32(N_DATA)
        z_s = acc_ref[1] / n_all * jnp.float32(N_DATA)
        c0 = jnp.float32(NCE_K * (1.0 / N_DATA))
        kiota = lax.broadcasted_iota(jnp.int32, (_LB, NCE_K + 1), 1)
        for ref, z, slot in ((outs_ref, z_s, 2), (outt_ref, z_t, 3)):
            x = ref[...] / z
            den = (x + c0) + jnp.float32(EPS)
            d1 = jnp.log(x / den)
            d0 = jnp.log(c0 / den)
            terms = jnp.where(kiota == 0, d1, d0)
            acc_ref[slot] = acc_ref[slot] + jnp.sum(terms)

        @pl.when(i == _NLB - 1)
        def _():
            loss_ref[...] = jnp.full(
                (1, 1), -(acc_ref[2] + acc_ref[3]) / jnp.float32(BATCH),
                jnp.float32)


def _loss(out_t2d, out_s2d):
    return pl.pallas_call(
        _loss_body,
        grid=(2, _NLB),
        in_specs=[
            pl.BlockSpec((_LB, NCE_K + 1), lambda p, i: (i, 0)),
            pl.BlockSpec((_LB, NCE_K + 1), lambda p, i: (i, 0)),
        ],
        out_specs=pl.BlockSpec((1, 1), lambda p, i: (0, 0)),
        out_shape=jax.ShapeDtypeStruct((1, 1), jnp.float32),
        scratch_shapes=[pltpu.SMEM((4,), jnp.float32)],
    )(out_t2d, out_s2d)


RPW = BATCH // NW


def _upd_one(rows_v, f_v, upd_v):
    def body(i, _):
        acc = jnp.zeros((L,), jnp.float32)
        for p_ in range(FEAT_DIM // L):
            sl = pl.ds(p_ * L, L)
            v = rows_v[i, sl] * jnp.float32(NCE_M) + \
                f_v[i, sl] * jnp.float32(1.0 - NCE_M)
            upd_v[i, sl] = v
            acc = acc + v * v
        s = jnp.sum(acc)
        sv = jnp.full((L,), s, jnp.float32)
        iv = plsc.bitcast(sv, jnp.int32)
        y = plsc.bitcast(jnp.int32(0x5F3759DF) - (iv >> 1), jnp.float32)
        for _n in range(4):
            y = y * (jnp.float32(1.5) - jnp.float32(0.5) * sv * y * y)
        for p_ in range(FEAT_DIM // L):
            sl = pl.ds(p_ * L, L)
            upd_v[i, sl] = upd_v[i, sl] * y
        return 0

    lax.fori_loop(0, RPW, body, 0)


def _upd_body(idx_hbm, win_hbm, fs_hbm, ft_hbm, m1_hbm, m2_hbm,
              new1_ref, new2_ref, idx_v, win_v, rows_v, f_v, upd_v, sem):
    base = _wid() * RPW
    pltpu.sync_copy(idx_hbm.at[pl.ds(base, RPW)], idx_v)
    pltpu.sync_copy(win_hbm.at[pl.ds(base, RPW)], win_v)
    for mem_hbm, f_hbm, new_ref in ((m1_hbm, fs_hbm, new1_ref),
                                    (m2_hbm, ft_hbm, new2_ref)):
        pltpu.async_copy(mem_hbm.at[idx_v], rows_v, sem).wait()
        pltpu.async_copy(f_hbm.at[win_v], f_v, sem).wait()
        _upd_one(rows_v, f_v, upd_v)
        pltpu.sync_copy(upd_v, new_ref.at[idx_v])


def _update(idx, winner, f_s, f_t, mem1, mem2, new1_ref, new2_ref):
    fn = pl.kernel(
        _upd_body,
        out_type=(),
        mesh=plsc.VectorSubcoreMesh(**_SC_MESH),
        compiler_params=pltpu.CompilerParams(needs_layout_passes=False),
        scratch_types=[
            pltpu.VMEM((RPW,), jnp.int32),
            pltpu.VMEM((RPW,), jnp.int32),
            pltpu.VMEM((RPW, FEAT_DIM), jnp.float32),
            pltpu.VMEM((RPW, FEAT_DIM), jnp.float32),
            pltpu.VMEM((RPW, FEAT_DIM), jnp.float32),
            pltpu.SemaphoreType.DMA,
        ],
    )
    fn(idx, winner, f_s, f_t, mem1, mem2, new1_ref, new2_ref)


def kernel(f_s, f_t, idx, contrast_idx, memory_v1, memory_v2):
    idx = idx.astype(jnp.int32)
    flat_idx = contrast_idx.reshape(-1).astype(jnp.int32)

    ar = jnp.arange(BATCH, dtype=jnp.int32)
    eq = idx[:, None] == idx[None, :]
    winner = jnp.max(jnp.where(eq, ar[None, :], -1), axis=1).astype(jnp.int32)

    st, cp1 = _scores(memory_v1, f_t)
    out_t_flat = _extract(flat_idx, st.reshape(N_DATA * _BB, FEAT_DIM))
    ss, cp2 = _scores(memory_v2, f_s)
    out_s_flat = _extract(flat_idx, ss.reshape(N_DATA * _BB, FEAT_DIM))
    loss = _loss(out_t_flat.reshape(BATCH, NCE_K + 1),
                 out_s_flat.reshape(BATCH, NCE_K + 1)).reshape((1,))

    new1_ref = jax.new_ref(cp1)
    new2_ref = jax.new_ref(cp2)
    _update(idx, winner, f_s, f_t, memory_v1, memory_v2, new1_ref, new2_ref)
    return loss, new1_ref[...], new2_ref[...]
